# baseline (speedup 1.0000x reference)
import functools

import jax
import jax.numpy as jnp
from jax.experimental import pallas as pl
from jax.experimental.pallas import tpu as pltpu
from jax.experimental.pallas import tpu_sc as plsc

Q = 1024
D = 1024
K = 100000
BS = 128
NB = 784
BN = 512
NBS = BN // BS
NSTEP = NB // NBS
KTOP = 50
KBLK = 56
SB = 16
NSUB = BS // SB
KSUB = 56
NEG = float("-inf")
IMAX = 2**31 - 1


def _scores_body(q_ref, k_ref, qn_ref, kn_ref, t_ref, s_ref, bm_ref, qs_ref):
    j = pl.program_id(0)

    @pl.when(j == 0)
    def _():
        qs_ref[...] = q_ref[...] / qn_ref[...]

    kn = k_ref[...] / kn_ref[...]
    s = jax.lax.dot_general(qs_ref[...], kn, (((1,), (1,)), ((), ())),
                            preferred_element_type=jnp.float32)
    s = s / t_ref[...]
    gcol = j * BN + jax.lax.broadcasted_iota(jnp.int32, (Q, BN), 1)
    s = jnp.where(gcol < K, s, NEG)
    parts = []
    maxes = []
    for i in range(NBS):
        blk = s[:, i * BS:(i + 1) * BS]
        parts.append(blk[None])
        maxes.append(jnp.max(blk, axis=1, keepdims=True)[None])
    s_ref[...] = jnp.concatenate(parts, axis=0)
    bm_ref[...] = jnp.concatenate(maxes, axis=0)


def _scores_call(queries, keys, qnorm, knorm, temp2d, interpret=False):
    return pl.pallas_call(
        _scores_body,
        grid=(NSTEP,),
        in_specs=[
            pl.BlockSpec((Q, D), lambda j: (0, 0)),
            pl.BlockSpec((BN, D), lambda j: (j, 0)),
            pl.BlockSpec((Q, 1), lambda j: (0, 0)),
            pl.BlockSpec((BN, 1), lambda j: (j, 0)),
            pl.BlockSpec((1, 1), lambda j: (0, 0)),
        ],
        out_specs=[
            pl.BlockSpec((NBS, Q, BS), lambda j: (j, 0, 0)),
            pl.BlockSpec((NBS, Q, 1), lambda j: (j, 0, 0)),
        ],
        out_shape=[
            jax.ShapeDtypeStruct((NB, Q, BS), jnp.float32),
            jax.ShapeDtypeStruct((NB, Q, 1), jnp.float32),
        ],
        scratch_shapes=[pltpu.VMEM((Q, D), jnp.float32)],
        interpret=interpret,
    )(queries, keys, qnorm, knorm, temp2d)


def _select_body(bm_ref, f_ref, b_ref):
    col = jax.lax.broadcasted_iota(jnp.int32, (Q, NB), 1)
    sel = jax.lax.broadcasted_iota(jnp.int32, (Q, KBLK), 1)

    def step(t, carry):
        x, acc_b = carry
        m = jnp.max(x, axis=1, keepdims=True)
        ci = jnp.min(jnp.where(x == m, col, IMAX), axis=1, keepdims=True)
        acc_b = jnp.where(sel == t, ci, acc_b)
        x = jnp.where(col == ci, NEG, x)
        return x, acc_b

    _, acc_b = jax.lax.fori_loop(
        0, KBLK, step, (bm_ref[...], jnp.zeros((Q, KBLK), jnp.int32)))
    r = jax.lax.broadcasted_iota(jnp.int32, (Q, KBLK), 0)
    f_ref[...] = acc_b * Q + r
    b_ref[...] = acc_b


def _select_call(bm, interpret=False):
    return pl.pallas_call(
        _select_body,
        in_specs=[pl.BlockSpec((Q, NB), lambda: (0, 0))],
        out_specs=[
            pl.BlockSpec((Q, KBLK), lambda: (0, 0)),
            pl.BlockSpec((Q, KBLK), lambda: (0, 0)),
        ],
        out_shape=[
            jax.ShapeDtypeStruct((Q, KBLK), jnp.int32),
            jax.ShapeDtypeStruct((Q, KBLK), jnp.int32),
        ],
        interpret=interpret,
    )(bm)


_MESH = dict(core_axis_name="c", subcore_axis_name="s")
_NWORK = 32
_CH = 128

_GROWS = Q * KBLK
_RPW = _GROWS // _NWORK
_NCH = _RPW // _CH


def _gather_body(tab_hbm, idx_hbm, out_hbm, idx_v, row_v, sem):
    w = jax.lax.axis_index("s") * 2 + jax.lax.axis_index("c")
    base = w * _RPW
    for c in range(_NCH):
        off = base + c * _CH
        pltpu.sync_copy(idx_hbm.at[pl.ds(off, _CH)], idx_v)
        pltpu.async_copy(tab_hbm.at[idx_v], row_v, sem).wait()
        pltpu.sync_copy(row_v, out_hbm.at[pl.ds(off, _CH)])


def _gather_call(table, fidx_flat):
    fn = functools.partial(
        pl.kernel,
        out_type=jax.ShapeDtypeStruct((_GROWS, BS), jnp.float32),
        mesh=plsc.VectorSubcoreMesh(**_MESH),
        scratch_types=[
            pltpu.VMEM((_CH,), jnp.int32),
            pltpu.VMEM((_CH, BS), jnp.float32),
            pltpu.SemaphoreType.DMA,
        ],
    )(_gather_body)
    return fn(table, fidx_flat)


_RB1 = 32


def _refine_body(c_ref, b_ref, f_ref, g_ref):
    v = c_ref[...].reshape(_RB1, KBLK, BS)
    maxes = []
    for i in range(NSUB):
        maxes.append(jnp.max(v[:, :, i * SB:(i + 1) * SB], axis=2, keepdims=True))
    x = jnp.concatenate(maxes, axis=2)
    isub = jax.lax.broadcasted_iota(jnp.int32, (_RB1, KBLK, NSUB), 2)
    base3 = b_ref[...] * BS + isub * SB
    s3 = jax.lax.broadcasted_iota(jnp.int32, (_RB1, KBLK, NSUB), 1) * NSUB + isub
    sel = jax.lax.broadcasted_iota(jnp.int32, (_RB1, KSUB, 1), 1)

    def step(t, carry):
        x, acc_s, acc_g = carry
        m = jnp.max(jnp.max(x, axis=2, keepdims=True), axis=1, keepdims=True)
        cb = jnp.where(x == m, base3, IMAX)
        gb = jnp.min(jnp.min(cb, axis=2, keepdims=True), axis=1, keepdims=True)
        chosen = base3 == gb
        si = jnp.min(jnp.min(jnp.where(chosen, s3, IMAX), axis=2, keepdims=True),
                     axis=1, keepdims=True)
        acc_s = jnp.where(sel == t, si, acc_s)
        acc_g = jnp.where(sel == t, gb, acc_g)
        x = jnp.where(chosen, NEG, x)
        return x, acc_s, acc_g

    _, acc_s, acc_g = jax.lax.fori_loop(
        0, KSUB, step,
        (x, jnp.zeros((_RB1, KSUB, 1), jnp.int32),
         jnp.zeros((_RB1, KSUB, 1), jnp.int32)))
    qg = pl.program_id(0) * _RB1 + jax.lax.broadcasted_iota(
        jnp.int32, (_RB1, KSUB, 1), 0)
    f_ref[...] = qg * (KBLK * NSUB) + acc_s
    g_ref[...] = acc_g


def _refine_call(cand, bidx3, interpret=False):
    return pl.pallas_call(
        _refine_body,
        grid=(Q // _RB1,),
        in_specs=[
            pl.BlockSpec((_RB1 * KBLK, BS), lambda r: (r, 0)),
            pl.BlockSpec((_RB1, KBLK, 1), lambda r: (r, 0, 0)),
        ],
        out_specs=[
            pl.BlockSpec((_RB1, KSUB, 1), lambda r: (r, 0, 0)),
            pl.BlockSpec((_RB1, KSUB, 1), lambda r: (r, 0, 0)),
        ],
        out_shape=[
            jax.ShapeDtypeStruct((Q, KSUB, 1), jnp.int32),
            jax.ShapeDtypeStruct((Q, KSUB, 1), jnp.int32),
        ],
        interpret=interpret,
    )(cand, bidx3)


_G2ROWS = Q * KSUB
_RPW2 = _G2ROWS // _NWORK
_NCH2 = _RPW2 // _CH
_PACK = BS // SB


def _gather2_body(tab_hbm, idx_hbm, out_hbm, idx_v, row_idx_v, stage_v, out_v, sem):
    w = jax.lax.axis_index("s") * 2 + jax.lax.axis_index("c")
    base = w * _RPW2

    @pl.loop(0, _NCH2)
    def _chunk(c):
        off = base + c * _CH
        pltpu.sync_copy(idx_hbm.at[pl.ds(off, _CH)], idx_v)
        for vi in range(_CH // 16):
            x = idx_v[pl.ds(vi * 16, 16)]
            row_idx_v[pl.ds(vi * 16, 16)] = jax.lax.shift_right_logical(x, 3)
        pltpu.async_copy(tab_hbm.at[row_idx_v], stage_v, sem).wait()
        iota = jax.lax.iota(jnp.int32, 16)
        for vi in range(_CH // 16):
            fi = idx_v[pl.ds(vi * 16, 16)]
            rows = vi * 16 + iota
            lane0 = jnp.bitwise_and(fi, NSUB - 1) * SB
            orow = jax.lax.shift_right_logical(rows, 3)
            ocol0 = jnp.bitwise_and(rows, _PACK - 1) * SB
            for l in range(SB):
                vals = plsc.load_gather(stage_v, [rows, lane0 + l])
                plsc.store_scatter(out_v, [orow, ocol0 + l], vals)
        ostart = pl.multiple_of(off // _PACK, _CH // _PACK)
        pltpu.sync_copy(out_v, out_hbm.at[pl.ds(ostart, _CH // _PACK)])


def _gather2_call(cand, fidx2_flat):
    fn = functools.partial(
        pl.kernel,
        out_type=jax.ShapeDtypeStruct((_G2ROWS // _PACK, BS), jnp.float32),
        mesh=plsc.VectorSubcoreMesh(**_MESH),
        compiler_params=pltpu.CompilerParams(needs_layout_passes=False),
        scratch_types=[
            pltpu.VMEM((_CH,), jnp.int32),
            pltpu.VMEM((_CH,), jnp.int32),
            pltpu.VMEM((_CH, BS), jnp.float32),
            pltpu.VMEM((_CH // _PACK, BS), jnp.float32),
            pltpu.SemaphoreType.DMA,
        ],
    )(_gather2_body)
    return fn(cand, fidx2_flat)


_RB2 = 128


_NPR = KSUB // _PACK


def _final_body(c_ref, g_ref, p_ref, i_ref):
    gb = g_ref[...]
    parts = [jnp.broadcast_to(gb[:, :, i:i + 1], (_RB2, _NPR, SB))
             for i in range(_PACK)]
    g = (jnp.concatenate(parts, axis=2)
         + (jax.lax.broadcasted_iota(jnp.int32, (_RB2, _NPR, BS), 2) & (SB - 1)))
    sel = jax.lax.broadcasted_iota(jnp.int32, (_RB2, KSUB, 1), 1)

    def step(t, carry):
        v, acc_v, acc_i = carry
        m = jnp.max(jnp.max(v, axis=2, keepdims=True), axis=1, keepdims=True)
        cand = jnp.where(v == m, g, IMAX)
        ci = jnp.min(jnp.min(cand, axis=2, keepdims=True), axis=1, keepdims=True)
        acc_v = jnp.where(sel == t, m, acc_v)
        acc_i = jnp.where(sel == t, ci, acc_i)
        v = jnp.where(g == ci, NEG, v)
        return v, acc_v, acc_i

    _, acc_v, acc_i = jax.lax.fori_loop(
        0, KTOP, step,
        (c_ref[...].reshape(_RB2, _NPR, BS),
         jnp.full((_RB2, KSUB, 1), NEG, jnp.float32),
         jnp.zeros((_RB2, KSUB, 1), jnp.int32)))
    e = jnp.exp(acc_v - acc_v[:, 0:1, :])
    p = e / jnp.sum(e, axis=1, keepdims=True)
    p_ref[...] = p[:, :KTOP, :]
    i_ref[...] = acc_i[:, :KTOP, :]


def _final_call(sub, gb3, interpret=False):
    return pl.pallas_call(
        _final_body,
        grid=(Q // _RB2,),
        in_specs=[
            pl.BlockSpec((_RB2 * _NPR, BS), lambda r: (r, 0)),
            pl.BlockSpec((_RB2, _NPR, _PACK), lambda r: (r, 0, 0)),
        ],
        out_specs=[
            pl.BlockSpec((_RB2, KTOP, 1), lambda r: (r, 0, 0)),
            pl.BlockSpec((_RB2, KTOP, 1), lambda r: (r, 0, 0)),
        ],
        out_shape=[
            jax.ShapeDtypeStruct((Q, KTOP, 1), jnp.float32),
            jax.ShapeDtypeStruct((Q, KTOP, 1), jnp.int32),
        ],
        interpret=interpret,
    )(sub, gb3)


def kernel(queries, keys, temp, k):
    del k
    qnorm = jnp.linalg.norm(queries, axis=-1, keepdims=True) + 1e-8
    knorm = jnp.linalg.norm(keys, axis=-1, keepdims=True) + 1e-8
    temp2d = jnp.asarray(temp, jnp.float32).reshape(1, 1)
    scores3, bmax3 = _scores_call(queries, keys, qnorm, knorm, temp2d)
    bm = bmax3.reshape(NB, Q).T
    fidx, bidx = _select_call(bm)
    table = scores3.reshape(NB * Q, BS)
    cand = _gather_call(table, fidx.reshape(_GROWS))
    fidx2, gb3 = _refine_call(cand, bidx.reshape(Q, KBLK, 1))
    sub = _gather2_call(cand, fidx2.reshape(_G2ROWS))
    probs3, idx3 = _final_call(sub, gb3.reshape(Q, _NPR, _PACK))
    return probs3.reshape(Q, KTOP), idx3.reshape(Q, KTOP)


# speedup vs baseline: 3.0923x; 3.0923x over previous
import functools

import jax
import jax.numpy as jnp
from jax.experimental import pallas as pl
from jax.experimental.pallas import tpu as pltpu
from jax.experimental.pallas import tpu_sc as plsc

Q = 1024
D = 1024
K = 100000
BS = 128
NB = 784
BN = 512
NBS = BN // BS
NSTEP = NB // NBS
KTOP = 50
KBLK = 56
SB = 16
NSUB = BS // SB
KSUB = 56
NEG = float("-inf")
IMAX = 2**31 - 1


def _scores_body(q_ref, k_ref, qn_ref, kn_ref, t_ref, s_ref, bm_ref, qs_ref):
    j = pl.program_id(0)

    @pl.when(j == 0)
    def _():
        qs_ref[...] = q_ref[...] / qn_ref[...]

    kn = k_ref[...] / kn_ref[...]
    s = jax.lax.dot_general(qs_ref[...], kn, (((1,), (1,)), ((), ())),
                            preferred_element_type=jnp.float32)
    s = s / t_ref[...]
    gcol = j * BN + jax.lax.broadcasted_iota(jnp.int32, (Q, BN), 1)
    s = jnp.where(gcol < K, s, NEG)
    parts = []
    maxes = []
    for i in range(NBS):
        blk = s[:, i * BS:(i + 1) * BS]
        parts.append(blk[None])
        maxes.append(jnp.max(blk, axis=1, keepdims=True)[None])
    s_ref[...] = jnp.concatenate(parts, axis=0)
    bm_ref[...] = jnp.concatenate(maxes, axis=0)


def _scores_call(queries, keys, qnorm, knorm, temp2d, interpret=False):
    return pl.pallas_call(
        _scores_body,
        grid=(NSTEP,),
        in_specs=[
            pl.BlockSpec((Q, D), lambda j: (0, 0)),
            pl.BlockSpec((BN, D), lambda j: (j, 0)),
            pl.BlockSpec((Q, 1), lambda j: (0, 0)),
            pl.BlockSpec((BN, 1), lambda j: (j, 0)),
            pl.BlockSpec((1, 1), lambda j: (0, 0)),
        ],
        out_specs=[
            pl.BlockSpec((NBS, Q, BS), lambda j: (j, 0, 0)),
            pl.BlockSpec((NBS, Q, 1), lambda j: (j, 0, 0)),
        ],
        out_shape=[
            jax.ShapeDtypeStruct((NB, Q, BS), jnp.float32),
            jax.ShapeDtypeStruct((NB, Q, 1), jnp.float32),
        ],
        scratch_shapes=[pltpu.VMEM((Q, D), jnp.float32)],
        interpret=interpret,
    )(queries, keys, qnorm, knorm, temp2d)


def _select_body(bm_ref, f_ref, b_ref):
    col = jax.lax.broadcasted_iota(jnp.int32, (Q, NB), 1)
    sel = jax.lax.broadcasted_iota(jnp.int32, (Q, KBLK), 1)

    def step(t, carry):
        x, acc_b = carry
        m = jnp.max(x, axis=1, keepdims=True)
        ci = jnp.min(jnp.where(x == m, col, IMAX), axis=1, keepdims=True)
        acc_b = jnp.where(sel == t, ci, acc_b)
        x = jnp.where(col == ci, NEG, x)
        return x, acc_b

    _, acc_b = jax.lax.fori_loop(
        0, KBLK, step, (bm_ref[...], jnp.zeros((Q, KBLK), jnp.int32)))
    r = jax.lax.broadcasted_iota(jnp.int32, (Q, KBLK), 0)
    f_ref[...] = acc_b * Q + r
    b_ref[...] = acc_b


def _select_call(bm, interpret=False):
    return pl.pallas_call(
        _select_body,
        in_specs=[pl.BlockSpec((Q, NB), lambda: (0, 0))],
        out_specs=[
            pl.BlockSpec((Q, KBLK), lambda: (0, 0)),
            pl.BlockSpec((Q, KBLK), lambda: (0, 0)),
        ],
        out_shape=[
            jax.ShapeDtypeStruct((Q, KBLK), jnp.int32),
            jax.ShapeDtypeStruct((Q, KBLK), jnp.int32),
        ],
        interpret=interpret,
    )(bm)


_MESH = dict(core_axis_name="c", subcore_axis_name="s")
_NWORK = 32
_CH = 128

_GROWS = Q * KBLK
_RPW = _GROWS // _NWORK
_NCH = _RPW // _CH


def _gather_body(tab_hbm, idx_hbm, out_hbm, idx_v, row_v, sem):
    w = jax.lax.axis_index("s") * 2 + jax.lax.axis_index("c")
    base = w * _RPW
    for c in range(_NCH):
        off = base + c * _CH
        pltpu.sync_copy(idx_hbm.at[pl.ds(off, _CH)], idx_v)
        pltpu.async_copy(tab_hbm.at[idx_v], row_v, sem).wait()
        pltpu.sync_copy(row_v, out_hbm.at[pl.ds(off, _CH)])


def _gather_call(table, fidx_flat):
    fn = functools.partial(
        pl.kernel,
        out_type=jax.ShapeDtypeStruct((_GROWS, BS), jnp.float32),
        mesh=plsc.VectorSubcoreMesh(**_MESH),
        scratch_types=[
            pltpu.VMEM((_CH,), jnp.int32),
            pltpu.VMEM((_CH, BS), jnp.float32),
            pltpu.SemaphoreType.DMA,
        ],
    )(_gather_body)
    return fn(table, fidx_flat)


_RB1 = 128
_NSS = KBLK * NSUB


def _refine_body(c_ref, b_ref, f_ref, g_ref):
    v = c_ref[...].reshape(_RB1, KBLK, BS)
    b2 = b_ref[...]
    xs, bases = [], []
    for i in range(NSUB):
        xs.append(jnp.max(v[:, :, i * SB:(i + 1) * SB], axis=2))
        bases.append(b2 * BS + i * SB)
    x = jnp.concatenate(xs, axis=1)
    base2 = jnp.concatenate(bases, axis=1)
    col = jax.lax.broadcasted_iota(jnp.int32, (_RB1, _NSS), 1)
    sel = jax.lax.broadcasted_iota(jnp.int32, (_RB1, KSUB), 1)

    def step(t, carry):
        x, acc_s, acc_g = carry
        m = jnp.max(x, axis=1, keepdims=True)
        gb = jnp.min(jnp.where(x == m, base2, IMAX), axis=1, keepdims=True)
        chosen = base2 == gb
        cc = jnp.min(jnp.where(chosen, col, IMAX), axis=1, keepdims=True)
        si = (cc % KBLK) * NSUB + cc // KBLK
        acc_s = jnp.where(sel == t, si, acc_s)
        acc_g = jnp.where(sel == t, gb, acc_g)
        x = jnp.where(chosen, NEG, x)
        return x, acc_s, acc_g

    _, acc_s, acc_g = jax.lax.fori_loop(
        0, KSUB, step,
        (x, jnp.zeros((_RB1, KSUB), jnp.int32),
         jnp.zeros((_RB1, KSUB), jnp.int32)))
    qg = pl.program_id(0) * _RB1 + jax.lax.broadcasted_iota(
        jnp.int32, (_RB1, KSUB), 0)
    f_ref[...] = qg * _NSS + acc_s
    g_ref[...] = acc_g


def _refine_call(cand, bidx2, interpret=False):
    return pl.pallas_call(
        _refine_body,
        grid=(Q // _RB1,),
        in_specs=[
            pl.BlockSpec((_RB1 * KBLK, BS), lambda r: (r, 0)),
            pl.BlockSpec((_RB1, KBLK), lambda r: (r, 0)),
        ],
        out_specs=[
            pl.BlockSpec((_RB1, KSUB), lambda r: (r, 0)),
            pl.BlockSpec((_RB1, KSUB), lambda r: (r, 0)),
        ],
        out_shape=[
            jax.ShapeDtypeStruct((Q, KSUB), jnp.int32),
            jax.ShapeDtypeStruct((Q, KSUB), jnp.int32),
        ],
        interpret=interpret,
    )(cand, bidx2)


_G2ROWS = Q * KSUB
_RPW2 = _G2ROWS // _NWORK
_NCH2 = _RPW2 // _CH
_PACK = BS // SB


def _gather2_body(tab_hbm, idx_hbm, out_hbm, idx_v, row_idx_v, stage_v, out_v, sem):
    w = jax.lax.axis_index("s") * 2 + jax.lax.axis_index("c")
    base = w * _RPW2

    @pl.loop(0, _NCH2)
    def _chunk(c):
        off = base + c * _CH
        pltpu.sync_copy(idx_hbm.at[pl.ds(off, _CH)], idx_v)
        for vi in range(_CH // 16):
            x = idx_v[pl.ds(vi * 16, 16)]
            row_idx_v[pl.ds(vi * 16, 16)] = jax.lax.shift_right_logical(x, 3)
        pltpu.async_copy(tab_hbm.at[row_idx_v], stage_v, sem).wait()
        iota = jax.lax.iota(jnp.int32, 16)
        for vi in range(_CH // 16):
            fi = idx_v[pl.ds(vi * 16, 16)]
            rows = vi * 16 + iota
            lane0 = jnp.bitwise_and(fi, NSUB - 1) * SB
            orow = jax.lax.shift_right_logical(rows, 3)
            ocol0 = jnp.bitwise_and(rows, _PACK - 1) * SB
            for l in range(SB):
                vals = plsc.load_gather(stage_v, [rows, lane0 + l])
                plsc.store_scatter(out_v, [orow, ocol0 + l], vals)
        ostart = pl.multiple_of(off // _PACK, _CH // _PACK)
        pltpu.sync_copy(out_v, out_hbm.at[pl.ds(ostart, _CH // _PACK)])


def _gather2_call(cand, fidx2_flat):
    fn = functools.partial(
        pl.kernel,
        out_type=jax.ShapeDtypeStruct((_G2ROWS // _PACK, BS), jnp.float32),
        mesh=plsc.VectorSubcoreMesh(**_MESH),
        compiler_params=pltpu.CompilerParams(needs_layout_passes=False),
        scratch_types=[
            pltpu.VMEM((_CH,), jnp.int32),
            pltpu.VMEM((_CH,), jnp.int32),
            pltpu.VMEM((_CH, BS), jnp.float32),
            pltpu.VMEM((_CH // _PACK, BS), jnp.float32),
            pltpu.SemaphoreType.DMA,
        ],
    )(_gather2_body)
    return fn(cand, fidx2_flat)


_RB2 = 128


_NPR = KSUB // _PACK


def _final_body(c_ref, g_ref, p_ref, i_ref):
    gb = g_ref[...]
    parts = [jnp.broadcast_to(gb[:, :, i:i + 1], (_RB2, _NPR, SB))
             for i in range(_PACK)]
    g = (jnp.concatenate(parts, axis=2)
         + (jax.lax.broadcasted_iota(jnp.int32, (_RB2, _NPR, BS), 2) & (SB - 1)))
    sel = jax.lax.broadcasted_iota(jnp.int32, (_RB2, KSUB), 1)

    def step(t, carry):
        v, acc_v, acc_i = carry
        m = jnp.max(jnp.max(v, axis=2, keepdims=True), axis=1, keepdims=True)
        cand = jnp.where(v == m, g, IMAX)
        ci = jnp.min(jnp.min(cand, axis=2, keepdims=True), axis=1, keepdims=True)
        acc_v = jnp.where(sel == t, m[:, 0, :], acc_v)
        acc_i = jnp.where(sel == t, ci[:, 0, :], acc_i)
        v = jnp.where(g == ci, NEG, v)
        return v, acc_v, acc_i

    _, acc_v, acc_i = jax.lax.fori_loop(
        0, KTOP, step,
        (c_ref[...].reshape(_RB2, _NPR, BS),
         jnp.full((_RB2, KSUB), NEG, jnp.float32),
         jnp.zeros((_RB2, KSUB), jnp.int32)))
    e = jnp.exp(acc_v - acc_v[:, 0:1])
    p = e / jnp.sum(e, axis=1, keepdims=True)
    p_ref[...] = p[:, :KTOP]
    i_ref[...] = acc_i[:, :KTOP]


def _final_call(sub, gb3, interpret=False):
    return pl.pallas_call(
        _final_body,
        grid=(Q // _RB2,),
        in_specs=[
            pl.BlockSpec((_RB2 * _NPR, BS), lambda r: (r, 0)),
            pl.BlockSpec((_RB2, _NPR, _PACK), lambda r: (r, 0, 0)),
        ],
        out_specs=[
            pl.BlockSpec((_RB2, KTOP), lambda r: (r, 0)),
            pl.BlockSpec((_RB2, KTOP), lambda r: (r, 0)),
        ],
        out_shape=[
            jax.ShapeDtypeStruct((Q, KTOP), jnp.float32),
            jax.ShapeDtypeStruct((Q, KTOP), jnp.int32),
        ],
        interpret=interpret,
    )(sub, gb3)


def kernel(queries, keys, temp, k):
    del k
    qnorm = jnp.linalg.norm(queries, axis=-1, keepdims=True) + 1e-8
    knorm = jnp.linalg.norm(keys, axis=-1, keepdims=True) + 1e-8
    temp2d = jnp.asarray(temp, jnp.float32).reshape(1, 1)
    scores3, bmax3 = _scores_call(queries, keys, qnorm, knorm, temp2d)
    bm = bmax3.reshape(NB, Q).T
    fidx, bidx = _select_call(bm)
    table = scores3.reshape(NB * Q, BS)
    cand = _gather_call(table, fidx.reshape(_GROWS))
    fidx2, gb2 = _refine_call(cand, bidx)
    sub = _gather2_call(cand, fidx2.reshape(_G2ROWS))
    return _final_call(sub, gb2.reshape(Q, _NPR, _PACK))


# speedup vs baseline: 3.2771x; 1.0597x over previous
import functools

import jax
import jax.numpy as jnp
from jax.experimental import pallas as pl
from jax.experimental.pallas import tpu as pltpu
from jax.experimental.pallas import tpu_sc as plsc

Q = 1024
D = 1024
K = 100000
BS = 128
NB = 784
BN = 512
NBS = BN // BS
NSTEP = NB // NBS
KTOP = 50
KBLK = 56
SB = 16
NSUB = BS // SB
KSUB = 56
NEG = float("-inf")
IMAX = 2**31 - 1


def _scores_body(q_ref, k_ref, qn_ref, kn_ref, t_ref, s_ref, bm_ref, qs_ref):
    j = pl.program_id(0)

    @pl.when(j == 0)
    def _():
        qs_ref[...] = q_ref[...] / qn_ref[...]

    kn = k_ref[...] / kn_ref[...]
    s = jax.lax.dot_general(qs_ref[...], kn, (((1,), (1,)), ((), ())),
                            preferred_element_type=jnp.float32)
    s = s / t_ref[...]
    gcol = j * BN + jax.lax.broadcasted_iota(jnp.int32, (Q, BN), 1)
    s = jnp.where(gcol < K, s, NEG)
    parts = []
    maxes = []
    for i in range(NBS):
        blk = s[:, i * BS:(i + 1) * BS]
        parts.append(blk[None])
        maxes.append(jnp.max(blk, axis=1, keepdims=True)[None])
    s_ref[...] = jnp.concatenate(parts, axis=0)
    bm_ref[...] = jnp.concatenate(maxes, axis=0)


def _scores_call(queries, keys, qnorm, knorm, temp2d, interpret=False):
    return pl.pallas_call(
        _scores_body,
        grid=(NSTEP,),
        in_specs=[
            pl.BlockSpec((Q, D), lambda j: (0, 0)),
            pl.BlockSpec((BN, D), lambda j: (j, 0)),
            pl.BlockSpec((Q, 1), lambda j: (0, 0)),
            pl.BlockSpec((BN, 1), lambda j: (j, 0)),
            pl.BlockSpec((1, 1), lambda j: (0, 0)),
        ],
        out_specs=[
            pl.BlockSpec((NBS, Q, BS), lambda j: (j, 0, 0)),
            pl.BlockSpec((NBS, Q, 1), lambda j: (j, 0, 0)),
        ],
        out_shape=[
            jax.ShapeDtypeStruct((NB, Q, BS), jnp.float32),
            jax.ShapeDtypeStruct((NB, Q, 1), jnp.float32),
        ],
        scratch_shapes=[pltpu.VMEM((Q, D), jnp.float32)],
        interpret=interpret,
    )(queries, keys, qnorm, knorm, temp2d)


def _select_body(bm_ref, f_ref, b_ref):
    col = jax.lax.broadcasted_iota(jnp.int32, (Q, NB), 1)
    sel = jax.lax.broadcasted_iota(jnp.int32, (Q, KBLK), 1)

    def step(t, carry):
        x, acc_b = carry
        m = jnp.max(x, axis=1, keepdims=True)
        ci = jnp.min(jnp.where(x == m, col, IMAX), axis=1, keepdims=True)
        acc_b = jnp.where(sel == t, ci, acc_b)
        x = jnp.where(col == ci, NEG, x)
        return x, acc_b

    _, acc_b = jax.lax.fori_loop(
        0, KBLK, step, (bm_ref[...], jnp.zeros((Q, KBLK), jnp.int32)),
        unroll=2)
    r = jax.lax.broadcasted_iota(jnp.int32, (Q, KBLK), 0)
    f_ref[...] = acc_b * Q + r
    b_ref[...] = acc_b


def _select_call(bm, interpret=False):
    return pl.pallas_call(
        _select_body,
        in_specs=[pl.BlockSpec((Q, NB), lambda: (0, 0))],
        out_specs=[
            pl.BlockSpec((Q, KBLK), lambda: (0, 0)),
            pl.BlockSpec((Q, KBLK), lambda: (0, 0)),
        ],
        out_shape=[
            jax.ShapeDtypeStruct((Q, KBLK), jnp.int32),
            jax.ShapeDtypeStruct((Q, KBLK), jnp.int32),
        ],
        interpret=interpret,
    )(bm)


_MESH = dict(core_axis_name="c", subcore_axis_name="s")
_NWORK = 32
_CH = 128

_GROWS = Q * KBLK
_RPW = _GROWS // _NWORK
_NCH = _RPW // _CH


def _gather_body(tab_hbm, idx_hbm, out_hbm, idx_v, row_v, sem):
    w = jax.lax.axis_index("s") * 2 + jax.lax.axis_index("c")
    base = w * _RPW
    for c in range(_NCH):
        off = base + c * _CH
        pltpu.sync_copy(idx_hbm.at[pl.ds(off, _CH)], idx_v)
        pltpu.async_copy(tab_hbm.at[idx_v], row_v, sem).wait()
        pltpu.sync_copy(row_v, out_hbm.at[pl.ds(off, _CH)])


def _gather_call(table, fidx_flat):
    fn = functools.partial(
        pl.kernel,
        out_type=jax.ShapeDtypeStruct((_GROWS, BS), jnp.float32),
        mesh=plsc.VectorSubcoreMesh(**_MESH),
        scratch_types=[
            pltpu.VMEM((_CH,), jnp.int32),
            pltpu.VMEM((_CH, BS), jnp.float32),
            pltpu.SemaphoreType.DMA,
        ],
    )(_gather_body)
    return fn(table, fidx_flat)


_RB1 = 128
_NSS = KBLK * NSUB


def _refine_body(c_ref, b_ref, f_ref, g_ref):
    v = c_ref[...].reshape(_RB1, KBLK, BS)
    b2 = b_ref[...]
    xs, bases = [], []
    for i in range(NSUB):
        xs.append(jnp.max(v[:, :, i * SB:(i + 1) * SB], axis=2))
        bases.append(b2 * BS + i * SB)
    x = jnp.concatenate(xs, axis=1)
    base2 = jnp.concatenate(bases, axis=1)
    col = jax.lax.broadcasted_iota(jnp.int32, (_RB1, _NSS), 1)
    sel = jax.lax.broadcasted_iota(jnp.int32, (_RB1, KSUB), 1)

    def step(t, carry):
        x, acc_s, acc_g = carry
        m = jnp.max(x, axis=1, keepdims=True)
        gb = jnp.min(jnp.where(x == m, base2, IMAX), axis=1, keepdims=True)
        chosen = base2 == gb
        cc = jnp.min(jnp.where(chosen, col, IMAX), axis=1, keepdims=True)
        si = (cc % KBLK) * NSUB + cc // KBLK
        acc_s = jnp.where(sel == t, si, acc_s)
        acc_g = jnp.where(sel == t, gb, acc_g)
        x = jnp.where(chosen, NEG, x)
        return x, acc_s, acc_g

    _, acc_s, acc_g = jax.lax.fori_loop(
        0, KSUB, step,
        (x, jnp.zeros((_RB1, KSUB), jnp.int32),
         jnp.zeros((_RB1, KSUB), jnp.int32)), unroll=2)
    qg = pl.program_id(0) * _RB1 + jax.lax.broadcasted_iota(
        jnp.int32, (_RB1, KSUB), 0)
    f_ref[...] = qg * _NSS + acc_s
    g_ref[...] = acc_g


def _refine_call(cand, bidx2, interpret=False):
    return pl.pallas_call(
        _refine_body,
        grid=(Q // _RB1,),
        in_specs=[
            pl.BlockSpec((_RB1 * KBLK, BS), lambda r: (r, 0)),
            pl.BlockSpec((_RB1, KBLK), lambda r: (r, 0)),
        ],
        out_specs=[
            pl.BlockSpec((_RB1, KSUB), lambda r: (r, 0)),
            pl.BlockSpec((_RB1, KSUB), lambda r: (r, 0)),
        ],
        out_shape=[
            jax.ShapeDtypeStruct((Q, KSUB), jnp.int32),
            jax.ShapeDtypeStruct((Q, KSUB), jnp.int32),
        ],
        interpret=interpret,
    )(cand, bidx2)


_G2ROWS = Q * KSUB
_RPW2 = _G2ROWS // _NWORK
_NCH2 = _RPW2 // _CH
_PACK = BS // SB


def _gather2_body(tab_hbm, idx_hbm, out_hbm, idx_v, row_idx_v, stage_v, out_v, sem):
    w = jax.lax.axis_index("s") * 2 + jax.lax.axis_index("c")
    base = w * _RPW2

    @pl.loop(0, _NCH2)
    def _chunk(c):
        off = base + c * _CH
        pltpu.sync_copy(idx_hbm.at[pl.ds(off, _CH)], idx_v)
        for vi in range(_CH // 16):
            x = idx_v[pl.ds(vi * 16, 16)]
            row_idx_v[pl.ds(vi * 16, 16)] = jax.lax.shift_right_logical(x, 3)
        pltpu.async_copy(tab_hbm.at[row_idx_v], stage_v, sem).wait()
        iota = jax.lax.iota(jnp.int32, 16)
        for vi in range(_CH // 16):
            fi = idx_v[pl.ds(vi * 16, 16)]
            rows = vi * 16 + iota
            lane0 = jnp.bitwise_and(fi, NSUB - 1) * SB
            orow = jax.lax.shift_right_logical(rows, 3)
            ocol0 = jnp.bitwise_and(rows, _PACK - 1) * SB
            for l in range(SB):
                vals = plsc.load_gather(stage_v, [rows, lane0 + l])
                plsc.store_scatter(out_v, [orow, ocol0 + l], vals)
        ostart = pl.multiple_of(off // _PACK, _CH // _PACK)
        pltpu.sync_copy(out_v, out_hbm.at[pl.ds(ostart, _CH // _PACK)])


def _gather2_call(cand, fidx2_flat):
    fn = functools.partial(
        pl.kernel,
        out_type=jax.ShapeDtypeStruct((_G2ROWS // _PACK, BS), jnp.float32),
        mesh=plsc.VectorSubcoreMesh(**_MESH),
        compiler_params=pltpu.CompilerParams(needs_layout_passes=False),
        scratch_types=[
            pltpu.VMEM((_CH,), jnp.int32),
            pltpu.VMEM((_CH,), jnp.int32),
            pltpu.VMEM((_CH, BS), jnp.float32),
            pltpu.VMEM((_CH // _PACK, BS), jnp.float32),
            pltpu.SemaphoreType.DMA,
        ],
    )(_gather2_body)
    return fn(cand, fidx2_flat)


_RB2 = 128


_NPR = KSUB // _PACK


def _final_body(c_ref, g_ref, p_ref, i_ref):
    gb = g_ref[...]
    parts = [jnp.broadcast_to(gb[:, :, i:i + 1], (_RB2, _NPR, SB))
             for i in range(_PACK)]
    g = (jnp.concatenate(parts, axis=2)
         + (jax.lax.broadcasted_iota(jnp.int32, (_RB2, _NPR, BS), 2) & (SB - 1)))
    sel = jax.lax.broadcasted_iota(jnp.int32, (_RB2, KSUB), 1)

    def step(t, carry):
        v, acc_v, acc_i = carry
        m = jnp.max(jnp.max(v, axis=2, keepdims=True), axis=1, keepdims=True)
        cand = jnp.where(v == m, g, IMAX)
        ci = jnp.min(jnp.min(cand, axis=2, keepdims=True), axis=1, keepdims=True)
        acc_v = jnp.where(sel == t, m[:, 0, :], acc_v)
        acc_i = jnp.where(sel == t, ci[:, 0, :], acc_i)
        v = jnp.where(g == ci, NEG, v)
        return v, acc_v, acc_i

    _, acc_v, acc_i = jax.lax.fori_loop(
        0, KTOP, step,
        (c_ref[...].reshape(_RB2, _NPR, BS),
         jnp.full((_RB2, KSUB), NEG, jnp.float32),
         jnp.zeros((_RB2, KSUB), jnp.int32)), unroll=2)
    e = jnp.exp(acc_v - acc_v[:, 0:1])
    p = e / jnp.sum(e, axis=1, keepdims=True)
    p_ref[...] = p[:, :KTOP]
    i_ref[...] = acc_i[:, :KTOP]


def _final_call(sub, gb3, interpret=False):
    return pl.pallas_call(
        _final_body,
        grid=(Q // _RB2,),
        in_specs=[
            pl.BlockSpec((_RB2 * _NPR, BS), lambda r: (r, 0)),
            pl.BlockSpec((_RB2, _NPR, _PACK), lambda r: (r, 0, 0)),
        ],
        out_specs=[
            pl.BlockSpec((_RB2, KTOP), lambda r: (r, 0)),
            pl.BlockSpec((_RB2, KTOP), lambda r: (r, 0)),
        ],
        out_shape=[
            jax.ShapeDtypeStruct((Q, KTOP), jnp.float32),
            jax.ShapeDtypeStruct((Q, KTOP), jnp.int32),
        ],
        interpret=interpret,
    )(sub, gb3)


def kernel(queries, keys, temp, k):
    del k
    qnorm = jnp.linalg.norm(queries, axis=-1, keepdims=True) + 1e-8
    knorm = jnp.linalg.norm(keys, axis=-1, keepdims=True) + 1e-8
    temp2d = jnp.asarray(temp, jnp.float32).reshape(1, 1)
    scores3, bmax3 = _scores_call(queries, keys, qnorm, knorm, temp2d)
    bm = bmax3.reshape(NB, Q).T
    fidx, bidx = _select_call(bm)
    table = scores3.reshape(NB * Q, BS)
    cand = _gather_call(table, fidx.reshape(_GROWS))
    fidx2, gb2 = _refine_call(cand, bidx)
    sub = _gather2_call(cand, fidx2.reshape(_G2ROWS))
    return _final_call(sub, gb2.reshape(Q, _NPR, _PACK))
    return _final_call(sub, gb2.reshape(Q, _NPR, _PACK))
